# speedup vs baseline: 3.4954x; 3.4954x over previous
import functools

import jax
import jax.numpy as jnp
from jax import lax
from jax.experimental import pallas as pl
from jax.experimental.pallas import tpu as pltpu
from jax.experimental.pallas import tpu_sc as plsc

HID = 64
G4 = 4 * HID


def _sc_gather(table, idx_flat, *, chunk=400):
    V, D = table.shape
    R = idx_flat.shape[0]
    info = plsc.get_sparse_core_info()
    nw = info.num_cores * info.num_subcores
    assert R % nw == 0
    per_w = R // nw
    assert per_w % chunk == 0 and chunk % 8 == 0
    n_chunks = per_w // chunk
    mesh = plsc.VectorSubcoreMesh(core_axis_name="c", subcore_axis_name="s")

    @functools.partial(
        pl.kernel,
        mesh=mesh,
        out_type=jax.ShapeDtypeStruct((R, D), table.dtype),
        scratch_types=[
            pltpu.VMEM((chunk,), jnp.int32),
            pltpu.VMEM((chunk, D), table.dtype),
            pltpu.SemaphoreType.DMA,
        ],
    )
    def gather_kernel(table_hbm, idx_hbm, out_hbm, idx_v, rows_v, sem):
        wid = lax.axis_index("s") * info.num_cores + lax.axis_index("c")
        base = wid * per_w

        def body(g, carry):
            off = base + g * chunk
            pltpu.sync_copy(idx_hbm.at[pl.ds(off, chunk)], idx_v)
            pltpu.async_copy(table_hbm.at[idx_v], rows_v, sem).wait()
            pltpu.sync_copy(rows_v, out_hbm.at[pl.ds(off, chunk), :])
            return carry

        lax.fori_loop(0, n_chunks, body, 0)

    return gather_kernel(table, idx_flat)


def _lstm_body(g_ref, wf_ref, uf_ref, bf_ref, wb_ref, ub_ref, bb_ref, out_ref):
    blk = out_ref.shape[0]
    k = g_ref.shape[0]
    wf = wf_ref[...]
    uf = uf_ref[...]
    bf = bf_ref[...]
    wb = wb_ref[...]
    ub = ub_ref[...]
    bb = bb_ref[...]

    def step(t, carry):
        h_f, c_f, h_b, c_b, acc_f, acc_b = carry
        xf = g_ref[t]
        xb = g_ref[k - 1 - t]
        gf = (jnp.dot(xf, wf, preferred_element_type=jnp.float32)
              + jnp.dot(h_f, uf, preferred_element_type=jnp.float32) + bf)
        gb = (jnp.dot(xb, wb, preferred_element_type=jnp.float32)
              + jnp.dot(h_b, ub, preferred_element_type=jnp.float32) + bb)
        i_f = jax.nn.sigmoid(gf[:, 0:HID])
        f_f = jax.nn.sigmoid(gf[:, HID:2 * HID])
        g_f = jnp.tanh(gf[:, 2 * HID:3 * HID])
        o_f = jax.nn.sigmoid(gf[:, 3 * HID:4 * HID])
        c_f = f_f * c_f + i_f * g_f
        h_f = o_f * jnp.tanh(c_f)
        i_b = jax.nn.sigmoid(gb[:, 0:HID])
        f_b = jax.nn.sigmoid(gb[:, HID:2 * HID])
        g_b = jnp.tanh(gb[:, 2 * HID:3 * HID])
        o_b = jax.nn.sigmoid(gb[:, 3 * HID:4 * HID])
        c_b = f_b * c_b + i_b * g_b
        h_b = o_b * jnp.tanh(c_b)
        return (h_f, c_f, h_b, c_b, acc_f + h_f, acc_b + h_b)

    z = jnp.zeros((blk, HID), jnp.float32)
    _, _, _, _, acc_f, acc_b = lax.fori_loop(0, k, step, (z, z, z, z, z, z))
    out_ref[...] = jnp.concatenate([acc_f, acc_b], axis=1) * (1.0 / k)


def _tc_bilstm_mean(g_tmajor, wih_f, whh_f, b_f, wih_b, whh_b, b_b, *, blk=400):
    k, n, d = g_tmajor.shape
    assert n % blk == 0
    nb = n // blk
    return pl.pallas_call(
        _lstm_body,
        grid=(nb,),
        in_specs=[
            pl.BlockSpec((k, blk, d), lambda i: (0, i, 0)),
            pl.BlockSpec((d, G4), lambda i: (0, 0)),
            pl.BlockSpec((HID, G4), lambda i: (0, 0)),
            pl.BlockSpec((1, G4), lambda i: (0, 0)),
            pl.BlockSpec((d, G4), lambda i: (0, 0)),
            pl.BlockSpec((HID, G4), lambda i: (0, 0)),
            pl.BlockSpec((1, G4), lambda i: (0, 0)),
        ],
        out_specs=pl.BlockSpec((blk, 2 * HID), lambda i: (i, 0)),
        out_shape=jax.ShapeDtypeStruct((n, 2 * HID), jnp.float32),
    )(g_tmajor, wih_f, whh_f, b_f, wih_b, whh_b, b_b)


def _aggregate(x_src, idx, params):
    wih_f, whh_f, bih_f, bhh_f, wih_b, whh_b, bih_b, bhh_b = params
    n, k = idx.shape
    d = x_src.shape[1]
    idx_t = jnp.transpose(idx.astype(jnp.int32)).reshape(-1)
    g = _sc_gather(x_src, idx_t)
    g = g.reshape(k, n, d)
    return _tc_bilstm_mean(
        g,
        jnp.transpose(wih_f), jnp.transpose(whh_f), (bih_f + bhh_f).reshape(1, G4),
        jnp.transpose(wih_b), jnp.transpose(whh_b), (bih_b + bhh_b).reshape(1, G4),
    )


@jax.jit
def kernel(x_paper, x_author, idx_paper_to_author, idx_author_to_paper,
           p_wih_f, p_whh_f, p_bih_f, p_bhh_f, p_wih_b, p_whh_b, p_bih_b, p_bhh_b,
           a_wih_f, a_whh_f, a_bih_f, a_bhh_f, a_wih_b, a_whh_b, a_bih_b, a_bhh_b):
    p_params = (p_wih_f, p_whh_f, p_bih_f, p_bhh_f, p_wih_b, p_whh_b, p_bih_b, p_bhh_b)
    a_params = (a_wih_f, a_whh_f, a_bih_f, a_bhh_f, a_wih_b, a_whh_b, a_bih_b, a_bhh_b)
    out_author = _aggregate(x_paper, idx_paper_to_author, p_params)
    out_paper = _aggregate(x_author, idx_author_to_paper, a_params)
    return (out_author, out_paper)


# speedup vs baseline: 3.6937x; 1.0567x over previous
import functools

import jax
import jax.numpy as jnp
from jax import lax
from jax.experimental import pallas as pl
from jax.experimental.pallas import tpu as pltpu
from jax.experimental.pallas import tpu_sc as plsc

HID = 64
G4 = 4 * HID


def _sc_gather(table, idx_flat, *, chunk=400):
    V, D = table.shape
    R = idx_flat.shape[0]
    info = plsc.get_sparse_core_info()
    nw = info.num_cores * info.num_subcores
    assert R % nw == 0
    per_w = R // nw
    assert per_w % chunk == 0 and chunk % 8 == 0
    n_chunks = per_w // chunk
    mesh = plsc.VectorSubcoreMesh(core_axis_name="c", subcore_axis_name="s")

    @functools.partial(
        pl.kernel,
        mesh=mesh,
        out_type=jax.ShapeDtypeStruct((R, D), table.dtype),
        scratch_types=[
            pltpu.VMEM((chunk,), jnp.int32),
            pltpu.VMEM((chunk, D), table.dtype),
            pltpu.SemaphoreType.DMA,
        ],
    )
    def gather_kernel(table_hbm, idx_hbm, out_hbm, idx_v, rows_v, sem):
        wid = lax.axis_index("s") * info.num_cores + lax.axis_index("c")
        base = wid * per_w

        def body(g, carry):
            off = base + g * chunk
            pltpu.sync_copy(idx_hbm.at[pl.ds(off, chunk)], idx_v)
            pltpu.async_copy(table_hbm.at[idx_v], rows_v, sem).wait()
            pltpu.sync_copy(rows_v, out_hbm.at[pl.ds(off, chunk), :])
            return carry

        lax.fori_loop(0, n_chunks, body, 0)

    return gather_kernel(table, idx_flat)


def _sigmoid(x):
    return 0.5 * jnp.tanh(0.5 * x) + 0.5


def _lstm_body(g_ref, wf_ref, uf_ref, bf_ref, wb_ref, ub_ref, bb_ref, out_ref,
               pf_ref, pb_ref):
    blk = out_ref.shape[0]
    k = g_ref.shape[0]
    uf = uf_ref[...]
    ub = ub_ref[...]

    x_all = g_ref[...].reshape(k * blk, g_ref.shape[2])
    pf_ref[...] = (jnp.dot(x_all, wf_ref[...], preferred_element_type=jnp.float32)
                   + bf_ref[...]).reshape(k, blk, G4)
    pb_ref[...] = (jnp.dot(x_all, wb_ref[...], preferred_element_type=jnp.float32)
                   + bb_ref[...]).reshape(k, blk, G4)

    z = jnp.zeros((blk, HID), jnp.float32)
    h_f, c_f, h_b, c_b, acc_f, acc_b = z, z, z, z, z, z
    for t in range(k):
        gf = pf_ref[t] + jnp.dot(h_f, uf, preferred_element_type=jnp.float32)
        gb = pb_ref[k - 1 - t] + jnp.dot(h_b, ub, preferred_element_type=jnp.float32)
        i_f = _sigmoid(gf[:, 0:HID])
        f_f = _sigmoid(gf[:, HID:2 * HID])
        g_f = jnp.tanh(gf[:, 2 * HID:3 * HID])
        o_f = _sigmoid(gf[:, 3 * HID:4 * HID])
        c_f = f_f * c_f + i_f * g_f
        h_f = o_f * jnp.tanh(c_f)
        i_b = _sigmoid(gb[:, 0:HID])
        f_b = _sigmoid(gb[:, HID:2 * HID])
        g_b = jnp.tanh(gb[:, 2 * HID:3 * HID])
        o_b = _sigmoid(gb[:, 3 * HID:4 * HID])
        c_b = f_b * c_b + i_b * g_b
        h_b = o_b * jnp.tanh(c_b)
        acc_f = acc_f + h_f
        acc_b = acc_b + h_b
    out_ref[...] = jnp.concatenate([acc_f, acc_b], axis=1) * (1.0 / k)


def _tc_bilstm_mean(g_tmajor, wih_f, whh_f, b_f, wih_b, whh_b, b_b, *, blk=200):
    k, n, d = g_tmajor.shape
    assert n % blk == 0
    nb = n // blk
    return pl.pallas_call(
        _lstm_body,
        grid=(nb,),
        in_specs=[
            pl.BlockSpec((k, blk, d), lambda i: (0, i, 0)),
            pl.BlockSpec((d, G4), lambda i: (0, 0)),
            pl.BlockSpec((HID, G4), lambda i: (0, 0)),
            pl.BlockSpec((1, G4), lambda i: (0, 0)),
            pl.BlockSpec((d, G4), lambda i: (0, 0)),
            pl.BlockSpec((HID, G4), lambda i: (0, 0)),
            pl.BlockSpec((1, G4), lambda i: (0, 0)),
        ],
        out_specs=pl.BlockSpec((blk, 2 * HID), lambda i: (i, 0)),
        out_shape=jax.ShapeDtypeStruct((n, 2 * HID), jnp.float32),
        scratch_shapes=[
            pltpu.VMEM((k, blk, G4), jnp.float32),
            pltpu.VMEM((k, blk, G4), jnp.float32),
        ],
    )(g_tmajor, wih_f, whh_f, b_f, wih_b, whh_b, b_b)


@jax.jit
def kernel(x_paper, x_author, idx_paper_to_author, idx_author_to_paper,
           p_wih_f, p_whh_f, p_bih_f, p_bhh_f, p_wih_b, p_whh_b, p_bih_b, p_bhh_b,
           a_wih_f, a_whh_f, a_bih_f, a_bhh_f, a_wih_b, a_whh_b, a_bih_b, a_bhh_b):
    n, k = idx_paper_to_author.shape
    d = x_paper.shape[1]
    idx0 = jnp.transpose(idx_paper_to_author.astype(jnp.int32)).reshape(-1)
    idx1 = jnp.transpose(idx_author_to_paper.astype(jnp.int32)).reshape(-1)
    g0 = _sc_gather(x_paper, idx0).reshape(k, n, d)
    g1 = _sc_gather(x_author, idx1).reshape(k, n, d)
    out_author = _tc_bilstm_mean(
        g0,
        jnp.transpose(p_wih_f), jnp.transpose(p_whh_f), (p_bih_f + p_bhh_f).reshape(1, G4),
        jnp.transpose(p_wih_b), jnp.transpose(p_whh_b), (p_bih_b + p_bhh_b).reshape(1, G4),
    )
    out_paper = _tc_bilstm_mean(
        g1,
        jnp.transpose(a_wih_f), jnp.transpose(a_whh_f), (a_bih_f + a_bhh_f).reshape(1, G4),
        jnp.transpose(a_wih_b), jnp.transpose(a_whh_b), (a_bih_b + a_bhh_b).reshape(1, G4),
    )
    return (out_author, out_paper)
